# Initial kernel scaffold; baseline (speedup 1.0000x reference)
import jax
import jax.numpy as jnp
from jax.experimental import pallas as pl


def kernel(node_latents, mesh_edge_latents, senders, receivers, W_e1, b_e1, W_e2, b_e2, g_e, beta_e, W_n1, b_n1, W_n2, b_n2, g_n, beta_n):
    raise NotImplementedError("write your pallas kernel here")


# speedup vs baseline: 3.4715x; 3.4715x over previous
import functools

import jax
import jax.numpy as jnp
from jax import lax
from jax.experimental import pallas as pl
from jax.experimental.pallas import tpu as pltpu
from jax.experimental.pallas import tpu_sc as plsc

N = 10000
E = 320000
L = 128

_NC = 2
_NS = 16
_NW = _NC * _NS

_CHUNK = 128
_NCHUNK = E // _CHUNK
_PER_W = -(-_NCHUNK // _NW)

_NPAD = 10240
_ROWS_PER_SUB = _NPAD // _NS


def _mesh():
    return plsc.VectorSubcoreMesh(core_axis_name="c", subcore_axis_name="s")


def _proj_body(n_ref, ws_ref, wr_ref, b_ref, s_out, r_out):
    x = n_ref[...]
    s_out[...] = jnp.dot(x, ws_ref[...], preferred_element_type=jnp.float32)
    r_out[...] = (
        jnp.dot(x, wr_ref[...], preferred_element_type=jnp.float32) + b_ref[...]
    )


def _proj(node, ws, wr, b):
    return pl.pallas_call(
        _proj_body,
        out_shape=(
            jax.ShapeDtypeStruct((N, L), jnp.float32),
            jax.ShapeDtypeStruct((N, L), jnp.float32),
        ),
    )(node, ws, wr, b)


def _gather_body(sproj, rproj, senders, receivers, out,
                 sidx, ridx, sbuf, rbuf, sem_s, sem_r):
    wid = lax.axis_index("s") * _NC + lax.axis_index("c")

    def chunk_body(j, carry):
        c = wid + _NW * j

        @pl.when(c < _NCHUNK)
        def _():
            base = c * _CHUNK
            pltpu.sync_copy(senders.at[pl.ds(base, _CHUNK)], sidx)
            pltpu.sync_copy(receivers.at[pl.ds(base, _CHUNK)], ridx)
            cp_s = pltpu.async_copy(sproj.at[sidx], sbuf, sem_s)
            cp_r = pltpu.async_copy(rproj.at[ridx], rbuf, sem_r)
            cp_s.wait()
            cp_r.wait()

            def row_body(r, carry2):
                for q in range(L // 16):
                    sl = pl.ds(q * 16, 16)
                    sbuf[r, sl] = sbuf[r, sl] + rbuf[r, sl]
                return carry2

            lax.fori_loop(0, _CHUNK, row_body, 0)
            pltpu.sync_copy(sbuf, out.at[pl.ds(base, _CHUNK)])

        return carry

    lax.fori_loop(0, _PER_W, chunk_body, 0)


def _gather_sum(sproj, rproj, senders, receivers):
    k = functools.partial(
        pl.kernel,
        out_type=jax.ShapeDtypeStruct((E, L), jnp.float32),
        mesh=_mesh(),
        scratch_types=[
            pltpu.VMEM((_CHUNK,), jnp.int32),
            pltpu.VMEM((_CHUNK,), jnp.int32),
            pltpu.VMEM((_CHUNK, L), jnp.float32),
            pltpu.VMEM((_CHUNK, L), jnp.float32),
            pltpu.SemaphoreType.DMA,
            pltpu.SemaphoreType.DMA,
        ],
    )(_gather_body)
    return k(sproj, rproj, senders, receivers)


def _edge_body(sr_ref, e_ref, wee_ref, we2_ref, b2_ref, g_ref, bet_ref,
               ne_ref, oe_ref):
    e = e_ref[...]
    h = sr_ref[...] + jnp.dot(e, wee_ref[...], preferred_element_type=jnp.float32)
    h = jnp.maximum(h, 0.0)
    h2 = jnp.dot(h, we2_ref[...], preferred_element_type=jnp.float32) + b2_ref[...]
    h2 = jnp.maximum(h2, 0.0)
    m = jnp.mean(h2, axis=-1, keepdims=True)
    cdev = h2 - m
    v = jnp.mean(cdev * cdev, axis=-1, keepdims=True)
    ne = cdev * lax.rsqrt(v + 1e-5) * g_ref[...] + bet_ref[...]
    ne_ref[...] = ne
    oe_ref[...] = ne + e


_EBLK = 2560


def _edge_mlp(sr, edge, wee, we2, b2, g, beta):
    grid = (E // _EBLK,)
    blk = lambda i: (i, 0)
    zero = lambda i: (0, 0)
    return pl.pallas_call(
        _edge_body,
        grid=grid,
        in_specs=[
            pl.BlockSpec((_EBLK, L), blk),
            pl.BlockSpec((_EBLK, L), blk),
            pl.BlockSpec((L, L), zero),
            pl.BlockSpec((L, L), zero),
            pl.BlockSpec((1, L), zero),
            pl.BlockSpec((1, L), zero),
            pl.BlockSpec((1, L), zero),
        ],
        out_specs=[
            pl.BlockSpec((_EBLK, L), blk),
            pl.BlockSpec((_EBLK, L), blk),
        ],
        out_shape=(
            jax.ShapeDtypeStruct((E, L), jnp.float32),
            jax.ShapeDtypeStruct((E, L), jnp.float32),
        ),
    )(sr, edge, wee, we2, b2, g, beta)


def _scatter_body(ne, receivers, p0, p1, ridx, buf, acc, sem):
    cid = lax.axis_index("c")
    sid = lax.axis_index("s")
    wid = sid * _NC + cid

    def zrow(r, carry):
        for q in range(L // 16):
            buf[r, pl.ds(q * 16, 16)] = jnp.zeros((16,), jnp.float32)
        return carry

    lax.fori_loop(0, _CHUNK, zrow, 0)
    for kk in range(_ROWS_PER_SUB // _CHUNK):
        pltpu.sync_copy(buf, acc.at[pl.ds(sid * _ROWS_PER_SUB + kk * _CHUNK, _CHUNK)])
    plsc.subcore_barrier()

    def chunk_body(j, carry):
        c = wid + _NW * j

        @pl.when(c < _NCHUNK)
        def _():
            base = c * _CHUNK
            pltpu.sync_copy(receivers.at[pl.ds(base, _CHUNK)], ridx)
            pltpu.sync_copy(ne.at[pl.ds(base, _CHUNK)], buf)
            pltpu.sync_copy(buf, acc.at[ridx], add=True)

        return carry

    lax.fori_loop(0, _PER_W, chunk_body, 0)
    plsc.subcore_barrier()

    for kk in range(_ROWS_PER_SUB // _CHUNK):
        r0 = sid * _ROWS_PER_SUB + kk * _CHUNK

        @pl.when(cid == 0)
        def _():
            pltpu.sync_copy(acc.at[pl.ds(r0, _CHUNK)], p0.at[pl.ds(r0, _CHUNK)])

        @pl.when(cid == 1)
        def _():
            pltpu.sync_copy(acc.at[pl.ds(r0, _CHUNK)], p1.at[pl.ds(r0, _CHUNK)])


def _scatter_add(ne, receivers):
    k = functools.partial(
        pl.kernel,
        out_type=(
            jax.ShapeDtypeStruct((_NPAD, L), jnp.float32),
            jax.ShapeDtypeStruct((_NPAD, L), jnp.float32),
        ),
        mesh=_mesh(),
        scratch_types=[
            pltpu.VMEM((_CHUNK,), jnp.int32),
            pltpu.VMEM((_CHUNK, L), jnp.float32),
            pltpu.VMEM_SHARED((_NPAD, L), jnp.float32),
            pltpu.SemaphoreType.DMA,
        ],
    )(_scatter_body)
    return k(ne, receivers)


def _node_body(n_ref, p0_ref, p1_ref, wna_ref, wnb_ref, b1_ref, w2_ref,
               b2_ref, g_ref, bet_ref, out_ref):
    x = n_ref[...]
    aggr = p0_ref[0:N, :] + p1_ref[0:N, :]
    h = (
        jnp.dot(x, wna_ref[...], preferred_element_type=jnp.float32)
        + jnp.dot(aggr, wnb_ref[...], preferred_element_type=jnp.float32)
        + b1_ref[...]
    )
    h = jnp.maximum(h, 0.0)
    h2 = jnp.dot(h, w2_ref[...], preferred_element_type=jnp.float32) + b2_ref[...]
    h2 = jnp.maximum(h2, 0.0)
    m = jnp.mean(h2, axis=-1, keepdims=True)
    cdev = h2 - m
    v = jnp.mean(cdev * cdev, axis=-1, keepdims=True)
    out_ref[...] = cdev * lax.rsqrt(v + 1e-5) * g_ref[...] + bet_ref[...] + x


def _node_mlp(node, p0, p1, wna, wnb, b1, w2, b2, g, beta):
    return pl.pallas_call(
        _node_body,
        out_shape=jax.ShapeDtypeStruct((N, L), jnp.float32),
    )(node, p0, p1, wna, wnb, b1, w2, b2, g, beta)


def kernel(node_latents, mesh_edge_latents, senders, receivers,
           W_e1, b_e1, W_e2, b_e2, g_e, beta_e,
           W_n1, b_n1, W_n2, b_n2, g_n, beta_n):
    node = node_latents.reshape(N, L)
    edge = mesh_edge_latents.reshape(E, L)
    snd = senders.astype(jnp.int32)
    rcv = receivers.astype(jnp.int32)

    ws = W_e1[0:L, :]
    wr = W_e1[L:2 * L, :]
    wee = W_e1[2 * L:3 * L, :]
    b1e = b_e1.reshape(1, L)

    sproj, rproj = _proj(node, ws, wr, b1e)
    sr = _gather_sum(sproj, rproj, snd, rcv)
    ne, new_edge = _edge_mlp(sr, edge, wee, W_e2, b_e2.reshape(1, L),
                             g_e.reshape(1, L), beta_e.reshape(1, L))
    p0, p1 = _scatter_add(ne, rcv)
    new_node = _node_mlp(node, p0, p1, W_n1[0:L, :], W_n1[L:2 * L, :],
                         b_n1.reshape(1, L), W_n2, b_n2.reshape(1, L),
                         g_n.reshape(1, L), beta_n.reshape(1, L))
    return new_node.reshape(1, N, L), new_edge.reshape(1, E, L)


# speedup vs baseline: 4.7969x; 1.3818x over previous
import functools

import jax
import jax.numpy as jnp
from jax import lax
from jax.experimental import pallas as pl
from jax.experimental.pallas import tpu as pltpu
from jax.experimental.pallas import tpu_sc as plsc

N = 10000
E = 320000
L = 128

_NC = 2
_NS = 16
_NW = _NC * _NS

_CHUNK = 128
_NCHUNK = E // _CHUNK
_PER_W = -(-_NCHUNK // _NW)

_NPAD = 10240
_ROWS_PER_SUB = _NPAD // _NS


def _mesh():
    return plsc.VectorSubcoreMesh(core_axis_name="c", subcore_axis_name="s")


def _proj_body(n_ref, ws_ref, wr_ref, b_ref, s_out, r_out):
    x = n_ref[...]
    s_out[...] = jnp.dot(x, ws_ref[...], preferred_element_type=jnp.float32)
    r_out[...] = (
        jnp.dot(x, wr_ref[...], preferred_element_type=jnp.float32) + b_ref[...]
    )


def _proj(node, ws, wr, b):
    return pl.pallas_call(
        _proj_body,
        out_shape=(
            jax.ShapeDtypeStruct((N, L), jnp.float32),
            jax.ShapeDtypeStruct((N, L), jnp.float32),
        ),
    )(node, ws, wr, b)


_NBUF = 3


def _gather_body(sproj, rproj, senders, receivers, out,
                 sidx, ridx, sbuf, rbuf, semg, semw):
    wid = lax.axis_index("s") * _NC + lax.axis_index("c")
    nw = jnp.where(wid < _NCHUNK - (_PER_W - 1) * _NW, _PER_W, _PER_W - 1)

    def issue(j, slot):
        c = wid + _NW * j
        base = c * _CHUNK
        pltpu.sync_copy(senders.at[pl.ds(base, _CHUNK)], sidx[slot])
        pltpu.sync_copy(receivers.at[pl.ds(base, _CHUNK)], ridx[slot])
        pltpu.async_copy(sproj.at[sidx[slot]], sbuf[slot], semg[slot])
        pltpu.async_copy(rproj.at[ridx[slot]], rbuf[slot], semg[slot])

    def process(j, slot):
        c = wid + _NW * j
        base = c * _CHUNK
        pltpu.make_async_copy(sproj.at[sidx[slot]], sbuf[slot], semg[slot]).wait()
        pltpu.make_async_copy(rproj.at[ridx[slot]], rbuf[slot], semg[slot]).wait()

        def row_body(r, carry2):
            for q in range(L // 16):
                sl = pl.ds(q * 16, 16)
                plsc.addupdate(sbuf[slot].at[r, sl], rbuf[slot][r, sl])
            return carry2

        lax.fori_loop(0, _CHUNK, row_body, 0)
        pltpu.async_copy(sbuf[slot], out.at[pl.ds(base, _CHUNK)], semw[slot])

    issue(0, 0)
    issue(1, 1)

    def triple_body(p, carry):
        for k in range(_NBUF):
            j = _NBUF * p + k
            jn = j + 2
            slot_n = (k + 2) % _NBUF

            @pl.when(jn < nw)
            def _():
                @pl.when(jn >= _NBUF)
                def _():
                    c_old = wid + _NW * (jn - _NBUF)
                    pltpu.make_async_copy(
                        sbuf[slot_n],
                        out.at[pl.ds((c_old) * _CHUNK, _CHUNK)],
                        semw[slot_n],
                    ).wait()

                issue(jn, slot_n)

            @pl.when(j < nw)
            def _():
                process(j, k)

        return carry

    lax.fori_loop(0, -(-_PER_W // _NBUF), triple_body, 0)

    for s in range(_NBUF):
        pltpu.make_async_copy(
            sbuf[s], out.at[pl.ds(0, _CHUNK)], semw[s]
        ).wait()


def _gather_sum(sproj, rproj, senders, receivers):
    k = functools.partial(
        pl.kernel,
        out_type=jax.ShapeDtypeStruct((E, L), jnp.float32),
        mesh=_mesh(),
        scratch_types=[
            [pltpu.VMEM((_CHUNK,), jnp.int32)] * _NBUF,
            [pltpu.VMEM((_CHUNK,), jnp.int32)] * _NBUF,
            [pltpu.VMEM((_CHUNK, L), jnp.float32)] * _NBUF,
            [pltpu.VMEM((_CHUNK, L), jnp.float32)] * _NBUF,
            [pltpu.SemaphoreType.DMA] * _NBUF,
            [pltpu.SemaphoreType.DMA] * _NBUF,
        ],
    )(_gather_body)
    return k(sproj, rproj, senders, receivers)


def _edge_body(sr_ref, e_ref, wee_ref, we2_ref, b2_ref, g_ref, bet_ref,
               ne_ref, oe_ref):
    e = e_ref[...]
    h = sr_ref[...] + jnp.dot(e, wee_ref[...], preferred_element_type=jnp.float32)
    h = jnp.maximum(h, 0.0)
    h2 = jnp.dot(h, we2_ref[...], preferred_element_type=jnp.float32) + b2_ref[...]
    h2 = jnp.maximum(h2, 0.0)
    m = jnp.mean(h2, axis=-1, keepdims=True)
    cdev = h2 - m
    v = jnp.mean(cdev * cdev, axis=-1, keepdims=True)
    ne = cdev * lax.rsqrt(v + 1e-5) * g_ref[...] + bet_ref[...]
    ne_ref[...] = ne
    oe_ref[...] = ne + e


_EBLK = 2560


def _edge_mlp(sr, edge, wee, we2, b2, g, beta):
    grid = (E // _EBLK,)
    blk = lambda i: (i, 0)
    zero = lambda i: (0, 0)
    return pl.pallas_call(
        _edge_body,
        grid=grid,
        in_specs=[
            pl.BlockSpec((_EBLK, L), blk),
            pl.BlockSpec((_EBLK, L), blk),
            pl.BlockSpec((L, L), zero),
            pl.BlockSpec((L, L), zero),
            pl.BlockSpec((1, L), zero),
            pl.BlockSpec((1, L), zero),
            pl.BlockSpec((1, L), zero),
        ],
        out_specs=[
            pl.BlockSpec((_EBLK, L), blk),
            pl.BlockSpec((_EBLK, L), blk),
        ],
        out_shape=(
            jax.ShapeDtypeStruct((E, L), jnp.float32),
            jax.ShapeDtypeStruct((E, L), jnp.float32),
        ),
    )(sr, edge, wee, we2, b2, g, beta)


def _scatter_body(ne, receivers, p0, p1, ridx, buf, acc, sem):
    cid = lax.axis_index("c")
    sid = lax.axis_index("s")
    wid = sid * _NC + cid

    def zrow(r, carry):
        for q in range(L // 16):
            buf[0][r, pl.ds(q * 16, 16)] = jnp.zeros((16,), jnp.float32)
        return carry

    lax.fori_loop(0, _CHUNK, zrow, 0)
    for kk in range(_ROWS_PER_SUB // _CHUNK):
        pltpu.sync_copy(
            buf[0], acc.at[pl.ds(sid * _ROWS_PER_SUB + kk * _CHUNK, _CHUNK)]
        )
    plsc.subcore_barrier()

    nw = jnp.where(wid < _NCHUNK - (_PER_W - 1) * _NW, _PER_W, _PER_W - 1)

    def issue(j, slot):
        base = (wid + _NW * j) * _CHUNK
        pltpu.sync_copy(receivers.at[pl.ds(base, _CHUNK)], ridx[slot])
        pltpu.async_copy(ne.at[pl.ds(base, _CHUNK)], buf[slot], sem[slot])

    def process(j, slot):
        base = (wid + _NW * j) * _CHUNK
        pltpu.make_async_copy(
            ne.at[pl.ds(base, _CHUNK)], buf[slot], sem[slot]
        ).wait()
        pltpu.sync_copy(buf[slot], acc.at[ridx[slot]], add=True)

    issue(0, 0)

    def pair_body(p, carry):
        for k in range(2):
            j = 2 * p + k
            jn = j + 1

            @pl.when(jn < nw)
            def _():
                issue(jn, 1 - k)

            @pl.when(j < nw)
            def _():
                process(j, k)

        return carry

    lax.fori_loop(0, -(-_PER_W // 2), pair_body, 0)
    plsc.subcore_barrier()

    for kk in range(_ROWS_PER_SUB // _CHUNK):
        r0 = sid * _ROWS_PER_SUB + kk * _CHUNK

        @pl.when(cid == 0)
        def _():
            pltpu.sync_copy(acc.at[pl.ds(r0, _CHUNK)], p0.at[pl.ds(r0, _CHUNK)])

        @pl.when(cid == 1)
        def _():
            pltpu.sync_copy(acc.at[pl.ds(r0, _CHUNK)], p1.at[pl.ds(r0, _CHUNK)])


def _scatter_add(ne, receivers):
    k = functools.partial(
        pl.kernel,
        out_type=(
            jax.ShapeDtypeStruct((_NPAD, L), jnp.float32),
            jax.ShapeDtypeStruct((_NPAD, L), jnp.float32),
        ),
        mesh=_mesh(),
        scratch_types=[
            [pltpu.VMEM((_CHUNK,), jnp.int32)] * 2,
            [pltpu.VMEM((_CHUNK, L), jnp.float32)] * 2,
            pltpu.VMEM_SHARED((_NPAD, L), jnp.float32),
            [pltpu.SemaphoreType.DMA] * 2,
        ],
    )(_scatter_body)
    return k(ne, receivers)


def _node_body(n_ref, p0_ref, p1_ref, wna_ref, wnb_ref, b1_ref, w2_ref,
               b2_ref, g_ref, bet_ref, out_ref):
    x = n_ref[...]
    aggr = p0_ref[0:N, :] + p1_ref[0:N, :]
    h = (
        jnp.dot(x, wna_ref[...], preferred_element_type=jnp.float32)
        + jnp.dot(aggr, wnb_ref[...], preferred_element_type=jnp.float32)
        + b1_ref[...]
    )
    h = jnp.maximum(h, 0.0)
    h2 = jnp.dot(h, w2_ref[...], preferred_element_type=jnp.float32) + b2_ref[...]
    h2 = jnp.maximum(h2, 0.0)
    m = jnp.mean(h2, axis=-1, keepdims=True)
    cdev = h2 - m
    v = jnp.mean(cdev * cdev, axis=-1, keepdims=True)
    out_ref[...] = cdev * lax.rsqrt(v + 1e-5) * g_ref[...] + bet_ref[...] + x


def _node_mlp(node, p0, p1, wna, wnb, b1, w2, b2, g, beta):
    return pl.pallas_call(
        _node_body,
        out_shape=jax.ShapeDtypeStruct((N, L), jnp.float32),
    )(node, p0, p1, wna, wnb, b1, w2, b2, g, beta)


def kernel(node_latents, mesh_edge_latents, senders, receivers,
           W_e1, b_e1, W_e2, b_e2, g_e, beta_e,
           W_n1, b_n1, W_n2, b_n2, g_n, beta_n):
    node = node_latents.reshape(N, L)
    edge = mesh_edge_latents.reshape(E, L)
    snd = senders.astype(jnp.int32)
    rcv = receivers.astype(jnp.int32)

    ws = W_e1[0:L, :]
    wr = W_e1[L:2 * L, :]
    wee = W_e1[2 * L:3 * L, :]
    b1e = b_e1.reshape(1, L)

    sproj, rproj = _proj(node, ws, wr, b1e)
    sr = _gather_sum(sproj, rproj, snd, rcv)
    ne, new_edge = _edge_mlp(sr, edge, wee, W_e2, b_e2.reshape(1, L),
                             g_e.reshape(1, L), beta_e.reshape(1, L))
    p0, p1 = _scatter_add(ne, rcv)
    new_node = _node_mlp(node, p0, p1, W_n1[0:L, :], W_n1[L:2 * L, :],
                         b_n1.reshape(1, L), W_n2, b_n2.reshape(1, L),
                         g_n.reshape(1, L), beta_n.reshape(1, L))
    return new_node.reshape(1, N, L), new_edge.reshape(1, E, L)
